# Initial kernel scaffold; baseline (speedup 1.0000x reference)
import jax
import jax.numpy as jnp
from jax.experimental import pallas as pl


def kernel(model_output_dist, target_sequence):
    raise NotImplementedError("write your pallas kernel here")


# speedup vs baseline: 5.3539x; 5.3539x over previous
import math

import jax
import jax.numpy as jnp
from jax.experimental import pallas as pl

_N = 4096
_V = 32000
_SMOOTHING = 0.1
_BR = 128

_S = _SMOOTHING / (_V - 3)
_CONF = 1.0 - _SMOOTHING
_C2 = (_V - 2) * _S * math.log(_S)
_C3 = (_V - 3) * _S * math.log(_S) + _CONF * math.log(_CONF)


def _loss_block(x_ref, t_ref, out_ref):
    i = pl.program_id(0)
    x = x_ref[...]
    t = t_ref[0]

    col = jax.lax.broadcasted_iota(jnp.int32, (_BR, _V), 1)
    row_ok = t != 0
    colmask = col >= 2
    dense = jnp.sum(jnp.where(row_ok & colmask, x, 0.0))

    gmask = (col == t) & (t >= 2)
    gathered = jnp.sum(jnp.where(gmask, x, 0.0))

    n_reg = jnp.sum((t >= 2).astype(jnp.float32))
    n_one = jnp.sum((t == 1).astype(jnp.float32))

    partial = jnp.reshape(_C3 * n_reg + _C2 * n_one
                          - _S * dense - (_CONF - _S) * gathered, (1, 1))

    @pl.when(i == 0)
    def _init():
        out_ref[...] = partial

    @pl.when(i != 0)
    def _acc():
        out_ref[...] += partial


def kernel(model_output_dist, target_sequence):
    n, v = model_output_dist.shape
    nb = n // _BR
    t = target_sequence.astype(jnp.int32).reshape(nb, _BR, 1)
    out = pl.pallas_call(
        _loss_block,
        grid=(nb,),
        in_specs=[
            pl.BlockSpec((_BR, v), lambda i: (i, 0)),
            pl.BlockSpec((1, _BR, 1), lambda i: (i, 0, 0)),
        ],
        out_specs=pl.BlockSpec((1, 1), lambda i: (0, 0)),
        out_shape=jax.ShapeDtypeStruct((1, 1), jnp.float32),
    )(model_output_dist, t)
    return out[0, 0]
